# Initial kernel scaffold; baseline (speedup 1.0000x reference)
import jax
import jax.numpy as jnp
from jax.experimental import pallas as pl


def kernel(x, mn_W0, mn_b0, mn_W1, mn_b1, me_W0, me_b0, me_W1, me_b1, c0_Wn, c0_bn, c0_We, c0_be, c0_Wa, c0_ba, c0_Wo, c0_bo, c1_Wn, c1_bn, c1_We, c1_be, c1_Wa, c1_ba, c1_Wo, c1_bo):
    raise NotImplementedError("write your pallas kernel here")


# speedup vs baseline: 11.8676x; 11.8676x over previous
import functools

import jax
import jax.numpy as jnp
from jax.experimental import pallas as pl
from jax.experimental.pallas import tpu as pltpu

_RADIUS = 0.1
_HID = 128
_CJ = 256
_BI_CANDIDATES = (80, 40, 8)


def _pick_bi(n):
    for b in _BI_CANDIDATES:
        if n % b == 0:
            return b
    return n


def _mask_he_kernel(n, sq_col_ref, sq_row_ref, embT_ref, emb_ref, x_ref,
                    ind_ref, mask_ref, hesum_ref, cnt_ref, occ_ref):
    g = pl.program_id(0)

    @pl.when(g == 0)
    def _init():
        hesum_ref[...] = jnp.zeros_like(hesum_ref)
        cnt_ref[...] = jnp.zeros_like(cnt_ref)

    gram = jnp.dot(emb_ref[...], embT_ref[...],
                   preferred_element_type=jnp.float32)
    d2 = sq_col_ref[...] + sq_row_ref[...] - 2.0 * gram
    d = jnp.sqrt(jnp.maximum(d2, 0.0))
    col = jax.lax.broadcasted_iota(jnp.int32, d.shape, 1)
    maskb = (d <= _RADIUS) & (col < n)
    mask_ref[...] = maskb.astype(jnp.int8)
    maskf = maskb.astype(jnp.float32)
    hesum_ref[...] += jax.lax.dot_general(
        maskf, x_ref[...], (((0,), (0,)), ((), ())),
        preferred_element_type=jnp.float32)
    cnt_ref[...] += jnp.sum(maskf, axis=0, keepdims=True)
    occ = jnp.dot(maskf, ind_ref[...], preferred_element_type=jnp.float32)
    occ_ref[...] = jnp.sum(occ, axis=0, keepdims=True).astype(jnp.int32)[None]


def _mask_he(node_emb, x, np_):
    n, d_in = x.shape
    bi = _pick_bi(n)
    grid = n // bi
    f32 = jnp.float32
    sq = jnp.sum(node_emb ** 2, axis=-1)
    pad = np_ - n
    nch = np_ // _CJ
    embTp = jnp.pad(node_emb.T, ((0, 0), (0, pad)))
    sq_rowp = jnp.pad(sq[None, :], ((0, 0), (0, pad)))
    ind = jnp.repeat(jnp.eye(nch, dtype=f32), _CJ, axis=0)
    return pl.pallas_call(
        functools.partial(_mask_he_kernel, n),
        grid=(grid,),
        in_specs=[
            pl.BlockSpec((bi, 1), lambda g: (g, 0)),
            pl.BlockSpec((1, np_), lambda g: (0, 0)),
            pl.BlockSpec((_HID, np_), lambda g: (0, 0)),
            pl.BlockSpec((bi, _HID), lambda g: (g, 0)),
            pl.BlockSpec((bi, d_in), lambda g: (g, 0)),
            pl.BlockSpec((np_, nch), lambda g: (0, 0)),
        ],
        out_specs=[
            pl.BlockSpec((bi, np_), lambda g: (g, 0)),
            pl.BlockSpec((np_, d_in), lambda g: (0, 0)),
            pl.BlockSpec((1, np_), lambda g: (0, 0)),
            pl.BlockSpec((1, 1, nch), lambda g: (g, 0, 0)),
        ],
        out_shape=[
            jax.ShapeDtypeStruct((n, np_), jnp.int8),
            jax.ShapeDtypeStruct((np_, d_in), f32),
            jax.ShapeDtypeStruct((1, np_), f32),
            jax.ShapeDtypeStruct((grid, 1, nch), jnp.int32),
        ],
        compiler_params=pltpu.CompilerParams(
            dimension_semantics=("arbitrary",)),
        interpret=False,
    )(sq[:, None], sq_rowp, embTp, node_emb, x, ind)


def _softmax_from_scores(s, maskb):
    m = jnp.max(jnp.where(maskb, s, -jnp.inf), axis=1, keepdims=True)
    m0 = jnp.where(jnp.isfinite(m), m, 0.0)
    p = jnp.where(maskb, jnp.exp(s - m0), 0.0)
    ssum = jnp.sum(p, axis=1, keepdims=True)
    return p / (ssum + 1e-16)


def _scores(xi, ejT, Wa_ref, ba_ref):
    bi = xi.shape[0]
    w = ejT.shape[1]
    acc = jnp.zeros((bi, w), dtype=jnp.float32)
    for k in range(_HID):
        acc += Wa_ref[0, k] * jnp.tanh(xi[:, k:k + 1] + ejT[k:k + 1, :])
    return acc + ba_ref[0, 0]


def _dense_h_kernel(xi_ref, ejT_ref, mask_ref, ej_ref, Wo_ref, bo_ref,
                    Wa_ref, ba_ref, h_ref):
    att = _softmax_from_scores(
        _scores(xi_ref[...], ejT_ref[...], Wa_ref, ba_ref),
        mask_ref[...] != 0)
    upd = jnp.dot(att, ej_ref[...], preferred_element_type=jnp.float32)
    out = jnp.dot(upd, Wo_ref[...], preferred_element_type=jnp.float32)
    h_ref[...] = jnp.maximum(out + bo_ref[...], 0.0)


def _dense_att_kernel(xi_ref, ejT_ref, mask_ref, Wa_ref, ba_ref, att_ref):
    att_ref[...] = _softmax_from_scores(
        _scores(xi_ref[...], ejT_ref[...], Wa_ref, ba_ref),
        mask_ref[...] != 0)


def _dense_conv(xi, ejTp, mask_i8, Wa, ba, ej=None, Wo=None, bo=None):
    n = xi.shape[0]
    np_ = ejTp.shape[1]
    bi = _pick_bi(n)
    grid = n // bi
    f32 = jnp.float32
    smem = pltpu.SMEM
    wa_row = Wa[:, 0][None, :]
    ba_s = ba.reshape(1, 1)
    common_specs = [
        pl.BlockSpec((bi, _HID), lambda g: (g, 0)),
        pl.BlockSpec((_HID, np_), lambda g: (0, 0)),
        pl.BlockSpec((bi, np_), lambda g: (g, 0)),
    ]
    scalar_specs = [
        pl.BlockSpec(memory_space=smem),
        pl.BlockSpec(memory_space=smem),
    ]
    if ej is not None:
        return pl.pallas_call(
            _dense_h_kernel,
            grid=(grid,),
            in_specs=common_specs + [
                pl.BlockSpec((np_, _HID), lambda g: (0, 0)),
                pl.BlockSpec((_HID, _HID), lambda g: (0, 0)),
                pl.BlockSpec((1, _HID), lambda g: (0, 0)),
            ] + scalar_specs,
            out_specs=pl.BlockSpec((bi, _HID), lambda g: (g, 0)),
            out_shape=jax.ShapeDtypeStruct((n, _HID), f32),
            compiler_params=pltpu.CompilerParams(
                dimension_semantics=("arbitrary",)),
            interpret=False,
        )(xi, ejTp, mask_i8, ej, Wo, bo[None, :], wa_row, ba_s)
    return pl.pallas_call(
        _dense_att_kernel,
        grid=(grid,),
        in_specs=common_specs + scalar_specs,
        out_specs=pl.BlockSpec((bi, np_), lambda g: (g, 0)),
        out_shape=jax.ShapeDtypeStruct((n, np_), f32),
        compiler_params=pltpu.CompilerParams(
            dimension_semantics=("arbitrary",)),
        interpret=False,
    )(xi, ejTp, mask_i8, wa_row, ba_s)


def _band_common(nch, bi, xi_ref, ejT0_ref, ejT1_ref, m0_ref, m1_ref,
                 Wa_ref, ba_ref):
    g = pl.program_id(0)
    c0 = (g * bi) // _CJ
    dup = (c0 + 1) > (nch - 1)
    mm0 = m0_ref[...] != 0
    mm1 = (m1_ref[...] != 0) & jnp.logical_not(dup)
    ejT = jnp.concatenate([ejT0_ref[...], ejT1_ref[...]], axis=1)
    s = _scores(xi_ref[...], ejT, Wa_ref, ba_ref)
    maskb = jnp.concatenate([mm0, mm1], axis=1)
    return _softmax_from_scores(s, maskb), dup


def _band_h_kernel(nch, bi, xi_ref, ejT0_ref, ejT1_ref, m0_ref, m1_ref,
                   ej0_ref, ej1_ref, Wo_ref, bo_ref, Wa_ref, ba_ref, h_ref):
    att, _ = _band_common(nch, bi, xi_ref, ejT0_ref, ejT1_ref, m0_ref, m1_ref,
                          Wa_ref, ba_ref)
    upd = (jnp.dot(att[:, :_CJ], ej0_ref[...],
                   preferred_element_type=jnp.float32)
           + jnp.dot(att[:, _CJ:], ej1_ref[...],
                     preferred_element_type=jnp.float32))
    out = jnp.dot(upd, Wo_ref[...], preferred_element_type=jnp.float32)
    h_ref[...] = jnp.maximum(out + bo_ref[...], 0.0)


def _band_att_kernel(nch, bi, z_ref, xi_ref, ejT0_ref, ejT1_ref, m0_ref,
                     m1_ref, Wa_ref, ba_ref, out_ref):
    att, dup = _band_common(nch, bi, xi_ref, ejT0_ref, ejT1_ref, m0_ref,
                            m1_ref, Wa_ref, ba_ref)
    t = pl.program_id(1)
    out_ref[...] = jnp.where((t == 1) & jnp.logical_not(dup),
                             att[:, _CJ:], att[:, :_CJ])


def _band_conv(xi, ejTp, mask_i8, Wa, ba, ej=None, Wo=None, bo=None):
    n = xi.shape[0]
    np_ = ejTp.shape[1]
    nch = np_ // _CJ
    bi = _pick_bi(n)
    grid = n // bi
    f32 = jnp.float32
    smem = pltpu.SMEM
    wa_row = Wa[:, 0][None, :]
    ba_s = ba.reshape(1, 1)

    def _c0(g):
        return (g * bi) // _CJ

    def _c1(g):
        c = _c0(g) + 1
        return jnp.minimum(c, nch - 1)

    scalar_specs = [
        pl.BlockSpec(memory_space=smem),
        pl.BlockSpec(memory_space=smem),
    ]
    if ej is not None:
        in_specs = [
            pl.BlockSpec((bi, _HID), lambda g: (g, 0)),
            pl.BlockSpec((_HID, _CJ), lambda g: (0, _c0(g))),
            pl.BlockSpec((_HID, _CJ), lambda g: (0, _c1(g))),
            pl.BlockSpec((bi, _CJ), lambda g: (g, _c0(g))),
            pl.BlockSpec((bi, _CJ), lambda g: (g, _c1(g))),
            pl.BlockSpec((_CJ, _HID), lambda g: (_c0(g), 0)),
            pl.BlockSpec((_CJ, _HID), lambda g: (_c1(g), 0)),
            pl.BlockSpec((_HID, _HID), lambda g: (0, 0)),
            pl.BlockSpec((1, _HID), lambda g: (0, 0)),
        ] + scalar_specs
        return pl.pallas_call(
            functools.partial(_band_h_kernel, nch, bi),
            grid=(grid,),
            in_specs=in_specs,
            out_specs=pl.BlockSpec((bi, _HID), lambda g: (g, 0)),
            out_shape=jax.ShapeDtypeStruct((n, _HID), f32),
            compiler_params=pltpu.CompilerParams(
                dimension_semantics=("arbitrary",)),
            interpret=False,
        )(xi, ejTp, ejTp, mask_i8, mask_i8, ej, ej, Wo, bo[None, :],
          wa_row, ba_s)

    zeros = jnp.zeros((n, np_), f32)
    in_specs = [
        pl.BlockSpec(memory_space=pl.ANY),
        pl.BlockSpec((bi, _HID), lambda g, t: (g, 0)),
        pl.BlockSpec((_HID, _CJ), lambda g, t: (0, _c0(g))),
        pl.BlockSpec((_HID, _CJ), lambda g, t: (0, _c1(g))),
        pl.BlockSpec((bi, _CJ), lambda g, t: (g, _c0(g))),
        pl.BlockSpec((bi, _CJ), lambda g, t: (g, _c1(g))),
        pl.BlockSpec(memory_space=smem),
        pl.BlockSpec(memory_space=smem),
    ]
    return pl.pallas_call(
        functools.partial(_band_att_kernel, nch, bi),
        grid=(grid, 2),
        in_specs=in_specs,
        out_specs=pl.BlockSpec(
            (bi, _CJ),
            lambda g, t: (g, jnp.minimum(_c0(g) + t, nch - 1))),
        out_shape=jax.ShapeDtypeStruct((n, np_), f32),
        input_output_aliases={0: 0},
        compiler_params=pltpu.CompilerParams(
            dimension_semantics=("arbitrary", "arbitrary")),
        interpret=False,
    )(zeros, xi, ejTp, ejTp, mask_i8, mask_i8, wa_row, ba_s)


def _mlp2(x, W0, b0, W1, b1):
    h = jnp.maximum(x @ W0 + b0, 0.0)
    return h @ W1 + b1


def kernel(x, mn_W0, mn_b0, mn_W1, mn_b1, me_W0, me_b0, me_W1, me_b1,
           c0_Wn, c0_bn, c0_We, c0_be, c0_Wa, c0_ba, c0_Wo, c0_bo,
           c1_Wn, c1_bn, c1_We, c1_be, c1_Wa, c1_ba, c1_Wo, c1_bo):
    n = x.shape[0]
    np_ = ((n + _CJ - 1) // _CJ) * _CJ
    nch = np_ // _CJ
    bi = _pick_bi(n)
    grid = n // bi
    node_emb = _mlp2(x, mn_W0, mn_b0, mn_W1, mn_b1)
    mask_i8, he_sum, cnt, occ3 = _mask_he(node_emb, x, np_)
    he = he_sum[:n] / jnp.maximum(cnt[0, :n], 1.0)[:, None]
    he_emb = _mlp2(he, me_W0, me_b0, me_W1, me_b1)
    e = jnp.concatenate([he, he_emb], axis=-1)
    xc = jnp.concatenate([x, node_emb], axis=-1)
    pad = np_ - n

    occ = occ3.reshape(grid, nch)
    gids = jnp.arange(grid)
    c0s = (gids * bi) // _CJ
    c1s = jnp.minimum(c0s + 1, nch - 1)
    band_cnt = (occ[gids, c0s]
                + jnp.where(c1s != c0s, occ[gids, c1s], 0))
    offband = jnp.sum(occ) - jnp.sum(band_cnt)

    xi0 = xc @ c0_Wn + c0_bn
    ej0 = jnp.pad(e @ c0_We + c0_be, ((0, pad), (0, 0)))
    ejT0 = ej0.T
    ej1w = jnp.pad(e @ c1_We + c1_be, ((0, pad), (0, 0)))
    ejT1 = ej1w.T

    def _fast(xi0, ej0, ejT0, ej1w, ejT1, mask_i8, h_args):
        (c0_Wa, c0_ba, c0_Wo, c0_bo, c1_Wn, c1_bn, c1_Wa, c1_ba) = h_args
        h = _band_conv(xi0, ejT0, mask_i8, c0_Wa, c0_ba,
                       ej=ej0, Wo=c0_Wo, bo=c0_bo)
        xi1 = h @ c1_Wn + c1_bn
        return _band_conv(xi1, ejT1, mask_i8, c1_Wa, c1_ba)

    def _slow(xi0, ej0, ejT0, ej1w, ejT1, mask_i8, h_args):
        (c0_Wa, c0_ba, c0_Wo, c0_bo, c1_Wn, c1_bn, c1_Wa, c1_ba) = h_args
        h = _dense_conv(xi0, ejT0, mask_i8, c0_Wa, c0_ba,
                        ej=ej0, Wo=c0_Wo, bo=c0_bo)
        xi1 = h @ c1_Wn + c1_bn
        return _dense_conv(xi1, ejT1, mask_i8, c1_Wa, c1_ba)

    h_args = (c0_Wa, c0_ba, c0_Wo, c0_bo, c1_Wn, c1_bn, c1_Wa, c1_ba)
    attp = jax.lax.cond(offband == 0, _fast, _slow,
                        xi0, ej0, ejT0, ej1w, ejT1, mask_i8, h_args)
    return attp[:, :n]


# speedup vs baseline: 13.4665x; 1.1347x over previous
import functools

import jax
import jax.numpy as jnp
from jax.experimental import pallas as pl
from jax.experimental.pallas import tpu as pltpu

_RADIUS = 0.1
_HID = 128
_CJ = 256
_BI_CANDIDATES = (80, 40, 8)


def _pick_bi(n):
    for b in _BI_CANDIDATES:
        if n % b == 0:
            return b
    return n


def _mask_he_kernel(n, sq_col_ref, sq_row_ref, embT_ref, emb_ref, x_ref,
                    ind_ref, mask_ref, hesum_ref, cnt_ref, occ_ref):
    g = pl.program_id(0)

    @pl.when(g == 0)
    def _init():
        hesum_ref[...] = jnp.zeros_like(hesum_ref)
        cnt_ref[...] = jnp.zeros_like(cnt_ref)

    gram = jnp.dot(emb_ref[...], embT_ref[...],
                   preferred_element_type=jnp.float32)
    d2 = sq_col_ref[...] + sq_row_ref[...] - 2.0 * gram
    d = jnp.sqrt(jnp.maximum(d2, 0.0))
    col = jax.lax.broadcasted_iota(jnp.int32, d.shape, 1)
    maskb = (d <= _RADIUS) & (col < n)
    mask_ref[...] = maskb.astype(jnp.int8)
    maskf = maskb.astype(jnp.float32)
    hesum_ref[...] += jax.lax.dot_general(
        maskf, x_ref[...], (((0,), (0,)), ((), ())),
        preferred_element_type=jnp.float32)
    cnt_ref[...] += jnp.sum(maskf, axis=0, keepdims=True)
    occ = jnp.dot(maskf, ind_ref[...], preferred_element_type=jnp.float32)
    occ_ref[...] = jnp.sum(occ, axis=0, keepdims=True).astype(jnp.int32)[None]


def _mask_he(node_emb, x, np_):
    n, d_in = x.shape
    bi = _pick_bi(n)
    grid = n // bi
    f32 = jnp.float32
    sq = jnp.sum(node_emb ** 2, axis=-1)
    pad = np_ - n
    nch = np_ // _CJ
    embTp = jnp.pad(node_emb.T, ((0, 0), (0, pad)))
    sq_rowp = jnp.pad(sq[None, :], ((0, 0), (0, pad)))
    ind = jnp.repeat(jnp.eye(nch, dtype=f32), _CJ, axis=0)
    return pl.pallas_call(
        functools.partial(_mask_he_kernel, n),
        grid=(grid,),
        in_specs=[
            pl.BlockSpec((bi, 1), lambda g: (g, 0)),
            pl.BlockSpec((1, np_), lambda g: (0, 0)),
            pl.BlockSpec((_HID, np_), lambda g: (0, 0)),
            pl.BlockSpec((bi, _HID), lambda g: (g, 0)),
            pl.BlockSpec((bi, d_in), lambda g: (g, 0)),
            pl.BlockSpec((np_, nch), lambda g: (0, 0)),
        ],
        out_specs=[
            pl.BlockSpec((bi, np_), lambda g: (g, 0)),
            pl.BlockSpec((np_, d_in), lambda g: (0, 0)),
            pl.BlockSpec((1, np_), lambda g: (0, 0)),
            pl.BlockSpec((1, 1, nch), lambda g: (g, 0, 0)),
        ],
        out_shape=[
            jax.ShapeDtypeStruct((n, np_), jnp.int8),
            jax.ShapeDtypeStruct((np_, d_in), f32),
            jax.ShapeDtypeStruct((1, np_), f32),
            jax.ShapeDtypeStruct((grid, 1, nch), jnp.int32),
        ],
        compiler_params=pltpu.CompilerParams(
            dimension_semantics=("arbitrary",)),
        interpret=False,
    )(sq[:, None], sq_rowp, embTp, node_emb, x, ind)


def _softmax_from_scores(s, maskb):
    m = jnp.max(jnp.where(maskb, s, -jnp.inf), axis=1, keepdims=True)
    m0 = jnp.where(jnp.isfinite(m), m, 0.0)
    p = jnp.where(maskb, jnp.exp(s - m0), 0.0)
    ssum = jnp.sum(p, axis=1, keepdims=True)
    return p / (ssum + 1e-16)


def _scores(xi, ejT, Wa_ref, ba_ref):
    bi = xi.shape[0]
    w = ejT.shape[1]
    acc = jnp.zeros((bi, w), dtype=jnp.float32)
    for k in range(_HID):
        acc += Wa_ref[0, k] * jnp.tanh(xi[:, k:k + 1] + ejT[k:k + 1, :])
    return acc + ba_ref[0, 0]


def _dense_h_kernel(xi_ref, ejT_ref, mask_ref, ej_ref, Wo_ref, bo_ref,
                    Wa_ref, ba_ref, h_ref):
    att = _softmax_from_scores(
        _scores(xi_ref[...], ejT_ref[...], Wa_ref, ba_ref),
        mask_ref[...] != 0)
    upd = jnp.dot(att, ej_ref[...], preferred_element_type=jnp.float32)
    out = jnp.dot(upd, Wo_ref[...], preferred_element_type=jnp.float32)
    h_ref[...] = jnp.maximum(out + bo_ref[...], 0.0)


def _dense_att_kernel(n, xi_ref, ejT_ref, mask_ref, Wa_ref, ba_ref, att_ref):
    att = _softmax_from_scores(
        _scores(xi_ref[...], ejT_ref[...], Wa_ref, ba_ref),
        mask_ref[...] != 0)
    att_ref[...] = att[:, :n]


def _dense_conv(xi, ejTp, mask_i8, Wa, ba, ej=None, Wo=None, bo=None):
    n = xi.shape[0]
    np_ = ejTp.shape[1]
    bi = _pick_bi(n)
    grid = n // bi
    f32 = jnp.float32
    smem = pltpu.SMEM
    wa_row = Wa[:, 0][None, :]
    ba_s = ba.reshape(1, 1)
    common_specs = [
        pl.BlockSpec((bi, _HID), lambda g: (g, 0)),
        pl.BlockSpec((_HID, np_), lambda g: (0, 0)),
        pl.BlockSpec((bi, np_), lambda g: (g, 0)),
    ]
    scalar_specs = [
        pl.BlockSpec(memory_space=smem),
        pl.BlockSpec(memory_space=smem),
    ]
    if ej is not None:
        return pl.pallas_call(
            _dense_h_kernel,
            grid=(grid,),
            in_specs=common_specs + [
                pl.BlockSpec((np_, _HID), lambda g: (0, 0)),
                pl.BlockSpec((_HID, _HID), lambda g: (0, 0)),
                pl.BlockSpec((1, _HID), lambda g: (0, 0)),
            ] + scalar_specs,
            out_specs=pl.BlockSpec((bi, _HID), lambda g: (g, 0)),
            out_shape=jax.ShapeDtypeStruct((n, _HID), f32),
            compiler_params=pltpu.CompilerParams(
                dimension_semantics=("arbitrary",)),
            interpret=False,
        )(xi, ejTp, mask_i8, ej, Wo, bo[None, :], wa_row, ba_s)
    return pl.pallas_call(
        functools.partial(_dense_att_kernel, n),
        grid=(grid,),
        in_specs=common_specs + scalar_specs,
        out_specs=pl.BlockSpec((bi, n), lambda g: (g, 0)),
        out_shape=jax.ShapeDtypeStruct((n, n), f32),
        compiler_params=pltpu.CompilerParams(
            dimension_semantics=("arbitrary",)),
        interpret=False,
    )(xi, ejTp, mask_i8, wa_row, ba_s)


def _band_common(nch, bi, xi_ref, ejT0_ref, ejT1_ref, m0_ref, m1_ref,
                 Wa_ref, ba_ref):
    g = pl.program_id(0)
    c0 = (g * bi) // _CJ
    dup = (c0 + 1) > (nch - 1)
    mm0 = m0_ref[...] != 0
    mm1 = (m1_ref[...] != 0) & jnp.logical_not(dup)
    ejT = jnp.concatenate([ejT0_ref[...], ejT1_ref[...]], axis=1)
    s = _scores(xi_ref[...], ejT, Wa_ref, ba_ref)
    maskb = jnp.concatenate([mm0, mm1], axis=1)
    return _softmax_from_scores(s, maskb), dup


def _band_h_kernel(nch, bi, xi_ref, ejT0_ref, ejT1_ref, m0_ref, m1_ref,
                   ej0_ref, ej1_ref, Wo_ref, bo_ref, Wa_ref, ba_ref, h_ref):
    att, _ = _band_common(nch, bi, xi_ref, ejT0_ref, ejT1_ref, m0_ref, m1_ref,
                          Wa_ref, ba_ref)
    upd = (jnp.dot(att[:, :_CJ], ej0_ref[...],
                   preferred_element_type=jnp.float32)
           + jnp.dot(att[:, _CJ:], ej1_ref[...],
                     preferred_element_type=jnp.float32))
    out = jnp.dot(upd, Wo_ref[...], preferred_element_type=jnp.float32)
    h_ref[...] = jnp.maximum(out + bo_ref[...], 0.0)


def _band_att_kernel(nch, bi, z_ref, xi_ref, ejT0_ref, ejT1_ref, m0_ref,
                     m1_ref, Wa_ref, ba_ref, out_ref):
    att, dup = _band_common(nch, bi, xi_ref, ejT0_ref, ejT1_ref, m0_ref,
                            m1_ref, Wa_ref, ba_ref)
    t = pl.program_id(1)
    out_ref[...] = jnp.where((t == 1) & jnp.logical_not(dup),
                             att[:, _CJ:], att[:, :_CJ])


def _band_conv(xi, ejTp, mask_i8, Wa, ba, ej=None, Wo=None, bo=None):
    n = xi.shape[0]
    np_ = ejTp.shape[1]
    nch = np_ // _CJ
    bi = _pick_bi(n)
    grid = n // bi
    f32 = jnp.float32
    smem = pltpu.SMEM
    wa_row = Wa[:, 0][None, :]
    ba_s = ba.reshape(1, 1)

    def _c0(g):
        return (g * bi) // _CJ

    def _c1(g):
        c = _c0(g) + 1
        return jnp.minimum(c, nch - 1)

    scalar_specs = [
        pl.BlockSpec(memory_space=smem),
        pl.BlockSpec(memory_space=smem),
    ]
    if ej is not None:
        in_specs = [
            pl.BlockSpec((bi, _HID), lambda g: (g, 0)),
            pl.BlockSpec((_HID, _CJ), lambda g: (0, _c0(g))),
            pl.BlockSpec((_HID, _CJ), lambda g: (0, _c1(g))),
            pl.BlockSpec((bi, _CJ), lambda g: (g, _c0(g))),
            pl.BlockSpec((bi, _CJ), lambda g: (g, _c1(g))),
            pl.BlockSpec((_CJ, _HID), lambda g: (_c0(g), 0)),
            pl.BlockSpec((_CJ, _HID), lambda g: (_c1(g), 0)),
            pl.BlockSpec((_HID, _HID), lambda g: (0, 0)),
            pl.BlockSpec((1, _HID), lambda g: (0, 0)),
        ] + scalar_specs
        return pl.pallas_call(
            functools.partial(_band_h_kernel, nch, bi),
            grid=(grid,),
            in_specs=in_specs,
            out_specs=pl.BlockSpec((bi, _HID), lambda g: (g, 0)),
            out_shape=jax.ShapeDtypeStruct((n, _HID), f32),
            compiler_params=pltpu.CompilerParams(
                dimension_semantics=("arbitrary",)),
            interpret=False,
        )(xi, ejTp, ejTp, mask_i8, mask_i8, ej, ej, Wo, bo[None, :],
          wa_row, ba_s)

    zeros = jnp.zeros((n, n), f32)
    in_specs = [
        pl.BlockSpec(memory_space=pl.ANY),
        pl.BlockSpec((bi, _HID), lambda g, t: (g, 0)),
        pl.BlockSpec((_HID, _CJ), lambda g, t: (0, _c0(g))),
        pl.BlockSpec((_HID, _CJ), lambda g, t: (0, _c1(g))),
        pl.BlockSpec((bi, _CJ), lambda g, t: (g, _c0(g))),
        pl.BlockSpec((bi, _CJ), lambda g, t: (g, _c1(g))),
        pl.BlockSpec(memory_space=smem),
        pl.BlockSpec(memory_space=smem),
    ]
    return pl.pallas_call(
        functools.partial(_band_att_kernel, nch, bi),
        grid=(grid, 2),
        in_specs=in_specs,
        out_specs=pl.BlockSpec(
            (bi, _CJ),
            lambda g, t: (g, jnp.minimum(_c0(g) + t, nch - 1))),
        out_shape=jax.ShapeDtypeStruct((n, n), f32),
        input_output_aliases={0: 0},
        compiler_params=pltpu.CompilerParams(
            dimension_semantics=("arbitrary", "arbitrary")),
        interpret=False,
    )(zeros, xi, ejTp, ejTp, mask_i8, mask_i8, wa_row, ba_s)


def _mlp2(x, W0, b0, W1, b1):
    h = jnp.maximum(x @ W0 + b0, 0.0)
    return h @ W1 + b1


def kernel(x, mn_W0, mn_b0, mn_W1, mn_b1, me_W0, me_b0, me_W1, me_b1,
           c0_Wn, c0_bn, c0_We, c0_be, c0_Wa, c0_ba, c0_Wo, c0_bo,
           c1_Wn, c1_bn, c1_We, c1_be, c1_Wa, c1_ba, c1_Wo, c1_bo):
    n = x.shape[0]
    np_ = ((n + _CJ - 1) // _CJ) * _CJ
    nch = np_ // _CJ
    bi = _pick_bi(n)
    grid = n // bi
    node_emb = _mlp2(x, mn_W0, mn_b0, mn_W1, mn_b1)
    mask_i8, he_sum, cnt, occ3 = _mask_he(node_emb, x, np_)
    he = he_sum[:n] / jnp.maximum(cnt[0, :n], 1.0)[:, None]
    he_emb = _mlp2(he, me_W0, me_b0, me_W1, me_b1)
    e = jnp.concatenate([he, he_emb], axis=-1)
    xc = jnp.concatenate([x, node_emb], axis=-1)
    pad = np_ - n

    occ = occ3.reshape(grid, nch)
    gids = jnp.arange(grid)
    c0s = (gids * bi) // _CJ
    c1s = jnp.minimum(c0s + 1, nch - 1)
    band_cnt = (occ[gids, c0s]
                + jnp.where(c1s != c0s, occ[gids, c1s], 0))
    offband = jnp.sum(occ) - jnp.sum(band_cnt)

    xi0 = xc @ c0_Wn + c0_bn
    ej0 = jnp.pad(e @ c0_We + c0_be, ((0, pad), (0, 0)))
    ejT0 = ej0.T
    ej1w = jnp.pad(e @ c1_We + c1_be, ((0, pad), (0, 0)))
    ejT1 = ej1w.T

    def _fast(xi0, ej0, ejT0, ej1w, ejT1, mask_i8, h_args):
        (c0_Wa, c0_ba, c0_Wo, c0_bo, c1_Wn, c1_bn, c1_Wa, c1_ba) = h_args
        h = _band_conv(xi0, ejT0, mask_i8, c0_Wa, c0_ba,
                       ej=ej0, Wo=c0_Wo, bo=c0_bo)
        xi1 = h @ c1_Wn + c1_bn
        return _band_conv(xi1, ejT1, mask_i8, c1_Wa, c1_ba)

    def _slow(xi0, ej0, ejT0, ej1w, ejT1, mask_i8, h_args):
        (c0_Wa, c0_ba, c0_Wo, c0_bo, c1_Wn, c1_bn, c1_Wa, c1_ba) = h_args
        h = _dense_conv(xi0, ejT0, mask_i8, c0_Wa, c0_ba,
                        ej=ej0, Wo=c0_Wo, bo=c0_bo)
        xi1 = h @ c1_Wn + c1_bn
        return _dense_conv(xi1, ejT1, mask_i8, c1_Wa, c1_ba)

    h_args = (c0_Wa, c0_ba, c0_Wo, c0_bo, c1_Wn, c1_bn, c1_Wa, c1_ba)
    return jax.lax.cond(offband == 0, _fast, _slow,
                        xi0, ej0, ejT0, ej1w, ejT1, mask_i8, h_args)
